# baseline (speedup 1.0000x reference)
import functools

import jax
import jax.numpy as jnp
from jax import lax
from jax.experimental import pallas as pl
from jax.experimental.pallas import tpu as pltpu
from jax.experimental.pallas import tpu_sc as plsc

_B = 8
_N = 2048
_T = _B * _N
_NW = 32
_TPW = _T // _NW
_W = _B * _N * _N
_WPW = _W // _NW
_NCH = _TPW // 128

_mesh = plsc.VectorSubcoreMesh(core_axis_name="c", subcore_axis_name="s")


@functools.partial(
    pl.kernel,
    out_type=jax.ShapeDtypeStruct((_W,), jnp.float32),
    mesh=_mesh,
    scratch_types=[
        pltpu.VMEM((_NCH, 128), jnp.int32),
        pltpu.VMEM((_NCH, 128), jnp.float32),
        pltpu.VMEM((_TPW,), jnp.float32),
        pltpu.VMEM((16,), jnp.float32),
        pltpu.SemaphoreType.DMA,
        pltpu.SemaphoreType.DMA,
    ],
)
def _sc_diag_add(cov_hbm, het_hbm, ns_hbm, out_hbm,
                 idx_v, vals_v, het_v, ns_v, sem_cp, sem_g):
    wid = lax.axis_index("s") * 2 + lax.axis_index("c")
    base_t = wid * _TPW
    base_w = wid * _WPW

    cp = pltpu.async_copy(
        cov_hbm.at[pl.ds(base_w, _WPW)],
        out_hbm.at[pl.ds(base_w, _WPW)],
        sem_cp,
    )

    pltpu.sync_copy(het_hbm.at[pl.ds(base_t, _TPW)], het_v)
    pltpu.sync_copy(ns_hbm, ns_v)

    iota = lax.iota(jnp.int32, 16)
    for k in range(_TPW // 16):
        t = base_t + k * 16 + iota
        idx_v[k // 8, pl.ds((k % 8) * 16, 16)] = (
            t * _N + jnp.bitwise_and(t, _N - 1))

    gathers = [
        pltpu.async_copy(cov_hbm.at[idx_v.at[j]], vals_v.at[j], sem_g)
        for j in range(_NCH)
    ]
    for g in gathers:
        g.wait()

    ens = jnp.exp(ns_v[...])
    for k in range(_TPW // 16):
        j, o = k // 8, (k % 8) * 16
        het16 = het_v[pl.ds(k * 16, 16)]
        vals_v[j, pl.ds(o, 16)] = (
            vals_v[j, pl.ds(o, 16)] + jnp.exp(het16) + ens)

    cp.wait()
    scatters = [
        pltpu.async_copy(vals_v.at[j], out_hbm.at[idx_v.at[j]], sem_g)
        for j in range(_NCH)
    ]
    for s in scatters:
        s.wait()


def kernel(cov, embeddings, noise_scale):
    het = embeddings[:, :, -1].reshape(_T)
    ns16 = jnp.broadcast_to(noise_scale, (16,))
    out = _sc_diag_add(cov.reshape(_W), het, ns16)
    return out.reshape(_B, _N, _N)


# speedup vs baseline: 15.6853x; 15.6853x over previous
import functools

import jax
import jax.numpy as jnp
from jax import lax
from jax.experimental import pallas as pl
from jax.experimental.pallas import tpu as pltpu
from jax.experimental.pallas import tpu_sc as plsc

_B = 8
_N = 2048
_T = _B * _N
_NW = 32
_TPW = _T // _NW
_W = _B * _N * _N
_NCH = _TPW // 128

_mesh = plsc.VectorSubcoreMesh(core_axis_name="c", subcore_axis_name="s")


def _stateful(refs):
    out_ref, het_ref, ns_ref = refs

    @pl.core_map(
        _mesh,
        scratch_shapes=[
            pltpu.VMEM((_NCH, 128), jnp.int32),
            pltpu.VMEM((_NCH, 128), jnp.float32),
            pltpu.VMEM((_TPW,), jnp.float32),
            pltpu.VMEM((16,), jnp.float32),
            pltpu.SemaphoreType.DMA,
        ],
    )
    def _body(idx_v, vals_v, het_v, ns_v, sem_g):
        wid = lax.axis_index("s") * 2 + lax.axis_index("c")
        base_t = wid * _TPW

        pltpu.sync_copy(het_ref.at[pl.ds(base_t, _TPW)], het_v)
        pltpu.sync_copy(ns_ref, ns_v)

        iota = lax.iota(jnp.int32, 16)
        for k in range(_TPW // 16):
            t = base_t + k * 16 + iota
            idx_v[k // 8, pl.ds((k % 8) * 16, 16)] = (
                t * _N + jnp.bitwise_and(t, _N - 1))

        gathers = [
            pltpu.async_copy(out_ref.at[idx_v.at[j]], vals_v.at[j], sem_g)
            for j in range(_NCH)
        ]
        for g in gathers:
            g.wait()

        ens = jnp.exp(ns_v[...])
        for k in range(_TPW // 16):
            j, o = k // 8, (k % 8) * 16
            het16 = het_v[pl.ds(k * 16, 16)]
            vals_v[j, pl.ds(o, 16)] = (
                vals_v[j, pl.ds(o, 16)] + jnp.exp(het16) + ens)

        scatters = [
            pltpu.async_copy(vals_v.at[j], out_ref.at[idx_v.at[j]], sem_g)
            for j in range(_NCH)
        ]
        for s in scatters:
            s.wait()


def kernel(cov, embeddings, noise_scale):
    het = embeddings[:, :, -1].reshape(_T)
    ns16 = jnp.broadcast_to(noise_scale, (16,))
    out, _, _ = pl.run_state(_stateful)((cov.reshape(_W), het, ns16))
    return out.reshape(_B, _N, _N)


# speedup vs baseline: 46.4598x; 2.9620x over previous
import jax
import jax.numpy as jnp
from jax.experimental import pallas as pl
from jax.experimental.pallas import tpu as pltpu

_B = 8
_N = 2048
_R = 1024
_NR = _N // _R


def _diag_body(ns_ref, cov_ref, het_ref, out_ref):
    r = pl.program_id(1)
    r0 = r * _R
    row = jax.lax.broadcasted_iota(jnp.int32, (_R, _N), 0)
    col = jax.lax.broadcasted_iota(jnp.int32, (_R, _N), 1)
    ens = jnp.exp(ns_ref[0])
    val = jnp.exp(het_ref[...]) + ens
    out_ref[0] = cov_ref[0] + jnp.where(col == row + r0, val, 0.0)


def kernel(cov, embeddings, noise_scale):
    het = embeddings[:, :, -1].reshape(_B * _N, 1)
    grid = (_B, _NR)
    out = pl.pallas_call(
        _diag_body,
        grid=grid,
        in_specs=[
            pl.BlockSpec(memory_space=pltpu.SMEM),
            pl.BlockSpec((1, _R, _N), lambda b, r: (b, r, 0)),
            pl.BlockSpec((_R, 1), lambda b, r: (b * _NR + r, 0)),
        ],
        out_specs=pl.BlockSpec((1, _R, _N), lambda b, r: (b, r, 0)),
        out_shape=jax.ShapeDtypeStruct((_B, _N, _N), jnp.float32),
    )(noise_scale, cov, het)
    return out
